# Initial kernel scaffold; baseline (speedup 1.0000x reference)
import jax
import jax.numpy as jnp
from jax.experimental import pallas as pl


def kernel(triples, entity_emb, relation_emb):
    raise NotImplementedError("write your pallas kernel here")


# speedup vs baseline: 1.1759x; 1.1759x over previous
import functools

import jax
import jax.numpy as jnp
from jax import lax
from jax.experimental import pallas as pl
from jax.experimental.pallas import tpu as pltpu
from jax.experimental.pallas import tpu_sc as plsc

BATCH = 4096
EMBED_DIM = 128
NUM_CORES = 2
NUM_SUBCORES = 16
NUM_WORKERS = NUM_CORES * NUM_SUBCORES
TRIPLES_PER_WORKER = BATCH // NUM_WORKERS
GROUPS = TRIPLES_PER_WORKER // 16


def _sc_score_kernel(heads_hbm, rels_hbm, tails_hbm, entity_hbm, relation_hbm,
                     out_hbm, hidx_v, ridx_v, tidx_v,
                     hrows_v, rrows_v, trows_v, scores_v, rot_v, sem):
    wid = lax.axis_index("s") * NUM_CORES + lax.axis_index("c")
    iota16 = lax.iota(jnp.int32, 16)

    idx_base = pl.multiple_of(wid * TRIPLES_PER_WORKER, 8)
    pltpu.sync_copy(heads_hbm.at[pl.ds(idx_base, TRIPLES_PER_WORKER)], hidx_v)
    pltpu.sync_copy(rels_hbm.at[pl.ds(idx_base, TRIPLES_PER_WORKER)], ridx_v)
    pltpu.sync_copy(tails_hbm.at[pl.ds(idx_base, TRIPLES_PER_WORKER)], tidx_v)

    cp_h = pltpu.async_copy(entity_hbm.at[hidx_v], hrows_v, sem)
    cp_r = pltpu.async_copy(relation_hbm.at[ridx_v], rrows_v, sem)
    cp_t = pltpu.async_copy(entity_hbm.at[tidx_v], trows_v, sem)
    cp_h.wait()
    cp_r.wait()
    cp_t.wait()

    for g in range(GROUPS):

        def triple_body(j, svec, g=g):
            i = g * 16 + j
            acc = jnp.zeros((16,), jnp.float32)
            for c in range(EMBED_DIM // 16):
                h = hrows_v[i, pl.ds(c * 16, 16)]
                r = rrows_v[i, pl.ds(c * 16, 16)]
                t = trows_v[i, pl.ds(c * 16, 16)]
                diff = h + r - t
                acc = acc + diff * diff
            v = acc
            for shift in (8, 4, 2, 1):
                rot_v[pl.ds(0, 16)] = v
                rot_v[pl.ds(16, 16)] = v
                v = v + rot_v[pl.ds(shift, 16)]
            return jnp.where(iota16 == j, v, svec)

        x = lax.fori_loop(0, 16, triple_body, jnp.zeros((16,), jnp.float32))

        x = x + 1e-12
        bits = lax.bitcast_convert_type(x, jnp.int32)
        bits = 0x5F3759DF - lax.shift_right_logical(bits, 1)
        y = lax.bitcast_convert_type(bits, jnp.float32)
        for _ in range(3):
            y = y * (1.5 - 0.5 * x * y * y)
        scores_v[pl.ds(g * 16, 16)] = -(x * y)

    out_base = pl.multiple_of(wid * TRIPLES_PER_WORKER, 8)
    pltpu.sync_copy(scores_v, out_hbm.at[pl.ds(out_base, TRIPLES_PER_WORKER)])


@jax.jit
def _sc_score(heads, rels, tails, entity_emb, relation_emb):
    mesh = plsc.VectorSubcoreMesh(core_axis_name="c", subcore_axis_name="s")
    return pl.kernel(
        _sc_score_kernel,
        out_type=jax.ShapeDtypeStruct((BATCH,), jnp.float32),
        mesh=mesh,
        scratch_types=[
            pltpu.VMEM((TRIPLES_PER_WORKER,), jnp.int32),
            pltpu.VMEM((TRIPLES_PER_WORKER,), jnp.int32),
            pltpu.VMEM((TRIPLES_PER_WORKER,), jnp.int32),
            pltpu.VMEM((TRIPLES_PER_WORKER, EMBED_DIM), jnp.float32),
            pltpu.VMEM((TRIPLES_PER_WORKER, EMBED_DIM), jnp.float32),
            pltpu.VMEM((TRIPLES_PER_WORKER, EMBED_DIM), jnp.float32),
            pltpu.VMEM((TRIPLES_PER_WORKER,), jnp.float32),
            pltpu.VMEM((32,), jnp.float32),
            pltpu.SemaphoreType.DMA,
        ],
    )(heads, rels, tails, entity_emb, relation_emb)


def kernel(triples, entity_emb, relation_emb):
    trip = triples.astype(jnp.int32)
    return _sc_score(trip[:, 0], trip[:, 1], trip[:, 2],
                     entity_emb, relation_emb)


# speedup vs baseline: 1.2921x; 1.0988x over previous
import functools

import jax
import jax.numpy as jnp
from jax import lax
from jax.experimental import pallas as pl
from jax.experimental.pallas import tpu as pltpu
from jax.experimental.pallas import tpu_sc as plsc

BATCH = 4096
EMBED_DIM = 128
NUM_CORES = 2
NUM_SUBCORES = 16
NUM_WORKERS = NUM_CORES * NUM_SUBCORES
TRIPLES_PER_WORKER = BATCH // NUM_WORKERS
GROUPS = TRIPLES_PER_WORKER // 16


def _sc_score_kernel(heads_hbm, rels_hbm, tails_hbm, entity_hbm, relation_hbm,
                     out_hbm, hidx_v, ridx_v, tidx_v,
                     hrows_v, rrows_v, trows_v, scores_v, rot_v, sem):
    wid = lax.axis_index("s") * NUM_CORES + lax.axis_index("c")
    iota16 = lax.iota(jnp.int32, 16)

    idx_base = pl.multiple_of(wid * TRIPLES_PER_WORKER, 8)
    pltpu.sync_copy(heads_hbm.at[pl.ds(idx_base, TRIPLES_PER_WORKER)], hidx_v)
    pltpu.sync_copy(rels_hbm.at[pl.ds(idx_base, TRIPLES_PER_WORKER)], ridx_v)
    pltpu.sync_copy(tails_hbm.at[pl.ds(idx_base, TRIPLES_PER_WORKER)], tidx_v)

    cp_h = pltpu.async_copy(entity_hbm.at[hidx_v], hrows_v, sem)
    cp_r = pltpu.async_copy(relation_hbm.at[ridx_v], rrows_v, sem)
    cp_t = pltpu.async_copy(entity_hbm.at[tidx_v], trows_v, sem)
    cp_h.wait()
    cp_r.wait()
    cp_t.wait()

    bitrev = (0, 8, 4, 12, 2, 10, 6, 14, 1, 9, 5, 13, 3, 11, 7, 15)
    m1 = iota16 < 8
    m2 = (iota16 & 4) == 0
    m3 = (iota16 & 2) == 0
    m4 = (iota16 & 1) == 0
    nslots = [0]

    def fold(v, shift):
        slot = nslots[0]
        nslots[0] = (slot + 1) % 32
        rot_v[slot, pl.ds(0, 16)] = v
        rot_v[slot, pl.ds(16, 16)] = v
        return v + rot_v[slot, pl.ds(shift, 16)]

    def group_body(g, carry):
        base = g * 16

        def leaf(l):
            i = base + bitrev[l]
            acc = None
            for c in range(EMBED_DIM // 16):
                h = hrows_v[i, pl.ds(c * 16, 16)]
                r = rrows_v[i, pl.ds(c * 16, 16)]
                t = trows_v[i, pl.ds(c * 16, 16)]
                d = h + r - t
                acc = d * d if acc is None else acc + d * d
            return acc

        a = [jnp.where(m1, fold(leaf(2 * p), 8), fold(leaf(2 * p + 1), 8))
             for p in range(8)]
        b = [jnp.where(m2, fold(a[2 * p], 4), fold(a[2 * p + 1], 12))
             for p in range(4)]
        c = [jnp.where(m3, fold(b[2 * p], 2), fold(b[2 * p + 1], 14))
             for p in range(2)]
        x = jnp.where(m4, fold(c[0], 1), fold(c[1], 15))

        x = x + 1e-12
        bits = lax.bitcast_convert_type(x, jnp.int32)
        bits = 0x5F3759DF - lax.shift_right_logical(bits, 1)
        y = lax.bitcast_convert_type(bits, jnp.float32)
        for _ in range(3):
            y = y * (1.5 - 0.5 * x * y * y)
        scores_v[pl.ds(base, 16)] = -(x * y)
        return carry

    lax.fori_loop(0, GROUPS, group_body, 0)

    out_base = pl.multiple_of(wid * TRIPLES_PER_WORKER, 8)
    pltpu.sync_copy(scores_v, out_hbm.at[pl.ds(out_base, TRIPLES_PER_WORKER)])


@jax.jit
def _sc_score(heads, rels, tails, entity_emb, relation_emb):
    mesh = plsc.VectorSubcoreMesh(core_axis_name="c", subcore_axis_name="s")
    return pl.kernel(
        _sc_score_kernel,
        out_type=jax.ShapeDtypeStruct((BATCH,), jnp.float32),
        mesh=mesh,
        scratch_types=[
            pltpu.VMEM((TRIPLES_PER_WORKER,), jnp.int32),
            pltpu.VMEM((TRIPLES_PER_WORKER,), jnp.int32),
            pltpu.VMEM((TRIPLES_PER_WORKER,), jnp.int32),
            pltpu.VMEM((TRIPLES_PER_WORKER, EMBED_DIM), jnp.float32),
            pltpu.VMEM((TRIPLES_PER_WORKER, EMBED_DIM), jnp.float32),
            pltpu.VMEM((TRIPLES_PER_WORKER, EMBED_DIM), jnp.float32),
            pltpu.VMEM((TRIPLES_PER_WORKER,), jnp.float32),
            pltpu.VMEM((32, 32), jnp.float32),
            pltpu.SemaphoreType.DMA,
        ],
    )(heads, rels, tails, entity_emb, relation_emb)


def kernel(triples, entity_emb, relation_emb):
    trip = triples.astype(jnp.int32)
    return _sc_score(trip[:, 0], trip[:, 1], trip[:, 2],
                     entity_emb, relation_emb)
